# Initial kernel scaffold; baseline (speedup 1.0000x reference)
import jax
import jax.numpy as jnp
from jax.experimental import pallas as pl


def kernel(x, W1, b1, W2, b2, W3, b3, g1, be1, g2, be2, g3, be3):
    raise NotImplementedError("write your pallas kernel here")


# speedup vs baseline: 9.2521x; 9.2521x over previous
import functools

import jax
import jax.numpy as jnp
from jax import lax
from jax.experimental import pallas as pl
from jax.experimental.pallas import tpu as pltpu
from jax.experimental.pallas import tpu_sc as plsc

B_SZ, NPTS, DIM, N_P = 8, 8192, 256, 32
S = B_SZ * NPTS
EPS = 1e-5
NEG = -3.0e38
_PREC = lax.Precision.DEFAULT


def _conv_body(x_ref, w_ref, b_ref, o_ref):
    o_ref[0] = jnp.dot(w_ref[...], x_ref[0], precision=_PREC,
                       preferred_element_type=jnp.float32) + b_ref[...]


def _bn_relu_conv_body(y_ref, m_ref, v_ref, g_ref, be_ref, w_ref, b_ref, o_ref):
    xhat = (y_ref[0] - m_ref[0]) / jnp.sqrt(v_ref[0] + EPS)
    u = jnp.maximum(xhat * g_ref[0] + be_ref[0], 0.0)
    o_ref[0] = jnp.dot(w_ref[...], u, precision=_PREC,
                       preferred_element_type=jnp.float32) + b_ref[...]


def _conv(x, W, cin, cout, bias):
    return pl.pallas_call(
        _conv_body, grid=(B_SZ,),
        in_specs=[pl.BlockSpec((1, cin, NPTS), lambda i: (i, 0, 0)),
                  pl.BlockSpec((cout, cin), lambda i: (0, 0)),
                  pl.BlockSpec((cout, 1), lambda i: (0, 0))],
        out_specs=pl.BlockSpec((1, cout, NPTS), lambda i: (i, 0, 0)),
        out_shape=jax.ShapeDtypeStruct((B_SZ, cout, NPTS), jnp.float32),
    )(x, W, bias)


def _bn_relu_conv(y, mean, var, g, be, W, cin, cout, bias):
    colb = pl.BlockSpec((1, cin, 1), lambda i: (0, 0, 0))
    return pl.pallas_call(
        _bn_relu_conv_body, grid=(B_SZ,),
        in_specs=[pl.BlockSpec((1, cin, NPTS), lambda i: (i, 0, 0)),
                  colb, colb, colb, colb,
                  pl.BlockSpec((cout, cin), lambda i: (0, 0)),
                  pl.BlockSpec((cout, 1), lambda i: (0, 0))],
        out_specs=pl.BlockSpec((1, cout, NPTS), lambda i: (i, 0, 0)),
        out_shape=jax.ShapeDtypeStruct((B_SZ, cout, NPTS), jnp.float32),
    )(y, mean, var, g, be, W, bias)


def _bn_topk_body(y_ref, m_ref, v_ref, g_ref, be_ref, h_ref, idx_ref, h_scr):
    xhat = (y_ref[0] - m_ref[0]) / jnp.sqrt(v_ref[0] + EPS)
    h0 = xhat * g_ref[0] + be_ref[0]
    h_ref[0] = h0
    h_scr[...] = h0
    ii = lax.broadcasted_iota(jnp.int32, (DIM, NPTS), 1)
    pp = lax.broadcasted_iota(jnp.int32, (DIM, N_P), 1)

    def body(p, acc):
        h = h_scr[...]
        m = jnp.max(h, axis=1, keepdims=True)
        sel = jnp.where(h >= m, ii, NPTS)
        ai = jnp.min(sel, axis=1, keepdims=True)
        h_scr[...] = jnp.where(ii == ai, NEG, h)
        return jnp.where(pp == p, ai, acc)

    acc0 = jnp.zeros((DIM, N_P), jnp.int32)
    idx_ref[...] = lax.fori_loop(0, N_P, body, acc0).reshape(1, DIM, N_P)


def _bn_topk(y, mean, var, g, be):
    colb = pl.BlockSpec((1, DIM, 1), lambda i: (0, 0, 0))
    return pl.pallas_call(
        _bn_topk_body, grid=(B_SZ,),
        in_specs=[pl.BlockSpec((1, DIM, NPTS), lambda i: (i, 0, 0)),
                  colb, colb, colb, colb],
        out_specs=[pl.BlockSpec((1, DIM, NPTS), lambda i: (i, 0, 0)),
                   pl.BlockSpec((1, DIM, N_P), lambda i: (i, 0, 0))],
        out_shape=[jax.ShapeDtypeStruct((B_SZ, DIM, NPTS), jnp.float32),
                   jax.ShapeDtypeStruct((B_SZ, DIM, N_P), jnp.int32)],
        scratch_shapes=[pltpu.VMEM((DIM, NPTS), jnp.float32)],
    )(y, mean, var, g, be)


def _moments(y):
    mean = jnp.mean(y, axis=(0, 2), keepdims=True)
    var = jnp.mean((y - mean) ** 2, axis=(0, 2), keepdims=True)
    return mean, var


def _sc_info():
    try:
        info = plsc.get_sparse_core_info()
        return info.num_cores, info.num_subcores
    except Exception:
        return 2, 16


def _gather_cube(h2d, idx_perm):
    NC, NS = _sc_info()
    NW = NC * NS
    rows_total = B_SZ * DIM
    rows_per_w = rows_total // NW
    mesh = plsc.VectorSubcoreMesh(core_axis_name="c", subcore_axis_name="s")

    @functools.partial(
        pl.kernel, mesh=mesh,
        out_type=jax.ShapeDtypeStruct((rows_total, NPTS), jnp.float32),
        scratch_types=[pltpu.VMEM((NPTS,), jnp.int32),
                       pltpu.VMEM((NPTS,), jnp.float32),
                       pltpu.VMEM((NPTS,), jnp.float32)],
        compiler_params=pltpu.CompilerParams(needs_layout_passes=False),
    )
    def gk(h_hbm, idx_hbm, out_hbm, idx_v, row_v, out_v):
        wid = lax.axis_index("s") * NC + lax.axis_index("c")
        b = wid // (NW // B_SZ)
        r0 = wid * rows_per_w
        pltpu.sync_copy(idx_hbm.at[b], idx_v)

        def do_row(i, _):
            r = r0 + i
            pltpu.sync_copy(h_hbm.at[r], row_v)

            def gat(j, _):
                iv = idx_v[pl.ds(j * 16, 16)]
                out_v[pl.ds(j * 16, 16)] = plsc.load_gather(row_v, [iv])
                return 0

            lax.fori_loop(0, NPTS // 16, gat, 0)
            pltpu.sync_copy(out_v, out_hbm.at[r])
            return 0

        lax.fori_loop(0, rows_per_w, do_row, 0)

    return gk(h2d, idx_perm)


def kernel(x, W1, b1, W2, b2, W3, b3, g1, be1, g2, be2, g3, be3):
    xp = jnp.pad(x, ((0, 0), (0, 5), (0, 0)))
    W1p = jnp.pad(W1, ((0, 0), (0, 5)))
    y1 = _conv(xp, W1p, 8, 64, b1[:, None])
    m1, v1 = _moments(y1)
    y2 = _bn_relu_conv(y1, m1, v1, g1[None, :, None], be1[None, :, None],
                       W2, 64, 128, b2[:, None])
    m2, v2 = _moments(y2)
    y3 = _bn_relu_conv(y2, m2, v2, g2[None, :, None], be2[None, :, None],
                       W3, 128, 256, b3[:, None])
    m3, v3 = _moments(y3)
    h, idx = _bn_topk(y3, m3, v3, g3[None, :, None], be3[None, :, None])
    idx_perm = jnp.transpose(idx, (0, 2, 1)).reshape(B_SZ, N_P * DIM)
    cube = _gather_cube(h.reshape(B_SZ * DIM, NPTS), idx_perm)
    return cube.reshape(B_SZ, DIM, N_P, DIM)
